# baseline (speedup 1.0000x reference)
import functools

import jax
import jax.numpy as jnp
from jax import lax
from jax.experimental import pallas as pl
from jax.experimental.pallas import tpu as pltpu
from jax.experimental.pallas import tpu_sc as plsc

N = 10000
E = 320000
NFEAT = 128
NHID = 128
NCLASS = 10
NMEM = 8
NGRAPH = 64

NP = 10240
NCHUNK = E // 128
KBASE = NCHUNK // 32
KREM = NCHUNK % 32
BLK = 400
NBLK = N // BLK

_NEG = -3.0e38


@functools.lru_cache(maxsize=None)
def _make_segsum(W, TW, dtype=jnp.float32):
    npass = TW // W
    rows_per_tile = NP // 16
    stage_rows = N // 16
    lanes = 16 * 4 // jnp.dtype(dtype).itemsize

    mesh = plsc.VectorSubcoreMesh(core_axis_name="c", subcore_axis_name="s")

    @functools.partial(
        pl.kernel,
        out_type=jax.ShapeDtypeStruct((npass, 2, NP, W), dtype),
        mesh=mesh,
        scratch_types=[
            pltpu.VMEM((KBASE + 1, 128), jnp.int32),
            pltpu.VMEM((KBASE + 1, 128), jnp.int32),
            pltpu.VMEM((2, 128, W), dtype),
            pltpu.VMEM_SHARED((NP, W), dtype),
            pltpu.VMEM_SHARED((NP, W), dtype),
            pltpu.SemaphoreType.DMA,
            pltpu.SemaphoreType.DMA,
        ],
        compiler_params=pltpu.CompilerParams(use_tc_tiling_on_sc=False),
    )
    def segsum(table_hbm, edge_hbm, out_hbm, src_v, dst_v, rows_v, tab_sh,
               acc_sh, sem0, sem1):
        cid = lax.axis_index("c")
        sid = lax.axis_index("s")
        tid = cid * 16 + sid
        base = tid * KBASE + jnp.minimum(tid, KREM)
        K = KBASE + jnp.where(tid < KREM, 1, 0)

        pltpu.sync_copy(edge_hbm.at[pl.ds(base, KBASE)],
                        src_v.at[pl.ds(0, KBASE)])
        pltpu.sync_copy(edge_hbm.at[pl.ds(NCHUNK + base, KBASE)],
                        dst_v.at[pl.ds(0, KBASE)])

        @pl.when(tid < KREM)
        def _extra():
            pltpu.sync_copy(edge_hbm.at[pl.ds(base + KBASE, 1)],
                            src_v.at[pl.ds(KBASE, 1)])
            pltpu.sync_copy(edge_hbm.at[pl.ds(NCHUNK + base + KBASE, 1)],
                            dst_v.at[pl.ds(KBASE, 1)])

        sems = (sem0, sem1)

        def _start(j, b):
            pltpu.async_copy(tab_sh.at[src_v.at[j]], rows_v.at[b], sems[b])

        def _wait(j, b):
            pltpu.make_async_copy(tab_sh.at[src_v.at[j]], rows_v.at[b],
                                  sems[b]).wait()

        def _scat(j, b):
            pltpu.sync_copy(rows_v.at[b], acc_sh.at[dst_v.at[j]], add=True)

        for p in range(npass):
            def _zrow(r, _):
                for j in range(W // lanes):
                    rows_v[0, r, pl.ds(j * lanes, lanes)] = jnp.zeros(
                        (lanes,), dtype)
                return 0
            lax.fori_loop(0, 128, _zrow, 0)
            for kk in range(rows_per_tile // 128):
                pltpu.sync_copy(
                    rows_v.at[0],
                    acc_sh.at[pl.ds(sid * rows_per_tile + kk * 128, 128)])
            pltpu.sync_copy(
                table_hbm.at[pl.ds(sid * stage_rows, stage_rows),
                             pl.ds(p * W, W)],
                tab_sh.at[pl.ds(sid * stage_rows, stage_rows)])
            plsc.subcore_barrier()

            _start(0, 0)

            def _body(i, _):
                jj = 2 * i
                _start(jj + 1, 1)
                _wait(jj, 0)
                _scat(jj, 0)

                @pl.when(jj + 2 < K)
                def _n0():
                    _start(jj + 2, 0)
                _wait(jj + 1, 1)
                _scat(jj + 1, 1)

                @pl.when(jj + 3 < K)
                def _n1():
                    _start(jj + 3, 1)
                return 0
            lax.fori_loop(0, KBASE // 2, _body, 0)

            @pl.when(K > KBASE)
            def _tail():
                _wait(KBASE, 0)
                _scat(KBASE, 0)

            plsc.subcore_barrier()
            pltpu.sync_copy(
                acc_sh.at[pl.ds(sid * rows_per_tile, rows_per_tile)],
                out_hbm.at[p, cid, pl.ds(sid * rows_per_tile, rows_per_tile)])

    return segsum


def _pre_body(x_ref, w_ref, b_ref, o_ref, ob_ref):
    r = jnp.dot(x_ref[...], w_ref[...],
                preferred_element_type=jnp.float32) + b_ref[...]
    o_ref[...] = r
    ob_ref[...] = r.astype(jnp.bfloat16)


def _tc_pre(x, W_pre, b_pre):
    return pl.pallas_call(
        _pre_body,
        grid=(NBLK,),
        in_specs=[
            pl.BlockSpec((BLK, NFEAT), lambda i: (i, 0)),
            pl.BlockSpec((NFEAT, NHID), lambda i: (0, 0)),
            pl.BlockSpec((1, NHID), lambda i: (0, 0)),
        ],
        out_specs=[pl.BlockSpec((BLK, NHID), lambda i: (i, 0)),
                   pl.BlockSpec((BLK, NHID), lambda i: (i, 0))],
        out_shape=[jax.ShapeDtypeStruct((N, NHID), jnp.float32),
                   jax.ShapeDtypeStruct((N, NHID), jnp.bfloat16)],
    )(x, W_pre, b_pre)


def _softmax8(t):
    m = jnp.max(t, axis=-1, keepdims=True)
    e = jnp.exp(t - m)
    return e / jnp.sum(e, axis=-1, keepdims=True)


def _layer0_body(h_ref, p0_ref, p1_ref,
                 w1_ref, b1_ref, w2_ref, b2_ref, mem_ref, o_ref):
    agg = (p0_ref[0, 0].astype(jnp.float32)
           + p1_ref[0, 0].astype(jnp.float32))
    z = jnp.concatenate([h_ref[...], agg], axis=1)
    z = jnp.maximum(jnp.dot(z, w1_ref[...],
                            preferred_element_type=jnp.float32) + b1_ref[...],
                    0.0)
    z = jnp.dot(z, w2_ref[...], preferred_element_type=jnp.float32) + b2_ref[...]
    z = jnp.maximum(z, 0.0)
    t = lax.dot_general(z, mem_ref[...], (((1,), (1,)), ((), ())),
                        preferred_element_type=jnp.float32)
    s = _softmax8(t)
    o_ref[...] = jnp.concatenate([s, jnp.zeros_like(s)], axis=1)


def _tc_layer0(h, p, W1, b1, W2, b2, mem):
    return pl.pallas_call(
        _layer0_body,
        grid=(NBLK,),
        in_specs=[
            pl.BlockSpec((BLK, NHID), lambda i: (i, 0)),
            pl.BlockSpec((1, 1, BLK, NHID), lambda i: (0, 0, i, 0)),
            pl.BlockSpec((1, 1, BLK, NHID), lambda i: (0, 1, i, 0)),
            pl.BlockSpec((2 * NHID, NHID), lambda i: (0, 0)),
            pl.BlockSpec((1, NHID), lambda i: (0, 0)),
            pl.BlockSpec((NHID, NHID), lambda i: (0, 0)),
            pl.BlockSpec((1, NHID), lambda i: (0, 0)),
            pl.BlockSpec((NMEM, NHID), lambda i: (0, 0)),
        ],
        out_specs=pl.BlockSpec((BLK, 16), lambda i: (i, 0)),
        out_shape=jax.ShapeDtypeStruct((N, 16), jnp.float32),
    )(h, p, p, W1, b1, W2, b2, mem)


def _pool(bat, x, max_ref, sum_ref):
    Pf = (lax.broadcasted_iota(jnp.int32, (BLK, NGRAPH), 1) == bat
          ).astype(jnp.float32)
    sum_ref[...] += lax.dot_general(Pf, x, (((0,), (0,)), ((), ())),
                                    preferred_element_type=jnp.float32)

    def _upd(g, _):
        m = jnp.max(jnp.where(bat == g, x, _NEG), axis=0, keepdims=True)
        max_ref[pl.ds(g, 1), :] = jnp.maximum(max_ref[pl.ds(g, 1), :], m)
        return 0
    lax.fori_loop(jnp.min(bat), jnp.max(bat) + 1, _upd, 0)
    return Pf


def _pool1_body(s0_ref, bat_ref, mem0_ref, maxo_ref, sumo_ref, cnto_ref,
                max_ref, sum_ref, cnt_ref):
    i = pl.program_id(0)

    @pl.when(i == 0)
    def _init():
        max_ref[...] = jnp.full_like(max_ref, _NEG)
        sum_ref[...] = jnp.zeros_like(sum_ref)
        cnt_ref[...] = jnp.zeros_like(cnt_ref)

    h1 = jnp.dot(s0_ref[...][:, :NMEM], mem0_ref[...],
                 preferred_element_type=jnp.float32)
    Pf = _pool(bat_ref[...], h1, max_ref, sum_ref)
    cnt_ref[...] += lax.dot_general(
        Pf, jnp.ones((BLK, 128), jnp.float32), (((0,), (0,)), ((), ())),
        preferred_element_type=jnp.float32)

    @pl.when(i == NBLK - 1)
    def _fin():
        maxo_ref[...] = max_ref[...]
        sumo_ref[...] = sum_ref[...]
        cnto_ref[...] = cnt_ref[...]


def _tc_pool1(s0p, batp, mem0):
    return pl.pallas_call(
        _pool1_body,
        grid=(NBLK,),
        in_specs=[
            pl.BlockSpec((BLK, 16), lambda i: (i, 0)),
            pl.BlockSpec((BLK, 1), lambda i: (i, 0)),
            pl.BlockSpec((NMEM, NHID), lambda i: (0, 0)),
        ],
        out_specs=[pl.BlockSpec((NGRAPH, NHID), lambda i: (0, 0))] * 3,
        out_shape=[jax.ShapeDtypeStruct((NGRAPH, NHID), jnp.float32)] * 3,
        scratch_shapes=[
            pltpu.VMEM((NGRAPH, NHID), jnp.float32),
            pltpu.VMEM((NGRAPH, NHID), jnp.float32),
            pltpu.VMEM((NGRAPH, NHID), jnp.float32),
        ],
    )(s0p, batp, mem0)


def _final_body(s0_ref, q0_ref, q1_ref, bat_ref, mem0_ref, w1_ref, b1_ref,
                w2_ref, b2_ref, mem1_ref, wl_ref, bl_ref,
                max1_ref, sum1_ref, cnt_ref, o_ref, max_ref, sum_ref):
    i = pl.program_id(0)

    @pl.when(i == 0)
    def _init():
        max_ref[...] = jnp.full_like(max_ref, _NEG)
        sum_ref[...] = jnp.zeros_like(sum_ref)

    s0 = s0_ref[...][:, :NMEM]
    a8 = (q0_ref[0, 0] + q1_ref[0, 0])[:, :NMEM]
    mem0 = mem0_ref[...]
    h1 = jnp.dot(s0, mem0, preferred_element_type=jnp.float32)
    agg1 = jnp.dot(a8, mem0, preferred_element_type=jnp.float32)
    z = jnp.concatenate([h1, agg1], axis=1)
    z = jnp.maximum(jnp.dot(z, w1_ref[...],
                            preferred_element_type=jnp.float32) + b1_ref[...],
                    0.0)
    z = jnp.dot(z, w2_ref[...], preferred_element_type=jnp.float32) + b2_ref[...]
    z = jnp.maximum(z, 0.0)
    t = lax.dot_general(z, mem1_ref[...], (((1,), (1,)), ((), ())),
                        preferred_element_type=jnp.float32)
    s1 = _softmax8(t)
    h2 = jnp.dot(s1, mem1_ref[...], preferred_element_type=jnp.float32)
    _pool(bat_ref[...], h2, max_ref, sum_ref)

    @pl.when(i == NBLK - 1)
    def _fin():
        def _fix(m):
            return jnp.where(m > _NEG * 0.5, m, 0.0)
        cnt = jnp.maximum(cnt_ref[...][:, 0:1], 1.0)
        gfeat = jnp.concatenate(
            [_fix(max1_ref[...]), _fix(max_ref[...]),
             sum1_ref[...] / cnt, sum_ref[...] / cnt], axis=1)
        logits = jnp.dot(gfeat, wl_ref[...],
                         preferred_element_type=jnp.float32) + bl_ref[...]
        m = jnp.max(logits, axis=-1, keepdims=True)
        lse = m + jnp.log(jnp.sum(jnp.exp(logits - m), axis=-1, keepdims=True))
        o_ref[...] = logits - lse


def _tc_final(s0p, q, batp, mem0, W1, b1, W2, b2, mem1, Wl, bl,
              max1, sum1, cnt1):
    return pl.pallas_call(
        _final_body,
        grid=(NBLK,),
        in_specs=[
            pl.BlockSpec((BLK, 16), lambda i: (i, 0)),
            pl.BlockSpec((1, 1, BLK, 16), lambda i: (0, 0, i, 0)),
            pl.BlockSpec((1, 1, BLK, 16), lambda i: (0, 1, i, 0)),
            pl.BlockSpec((BLK, 1), lambda i: (i, 0)),
            pl.BlockSpec((NMEM, NHID), lambda i: (0, 0)),
            pl.BlockSpec((2 * NHID, NHID), lambda i: (0, 0)),
            pl.BlockSpec((1, NHID), lambda i: (0, 0)),
            pl.BlockSpec((NHID, NHID), lambda i: (0, 0)),
            pl.BlockSpec((1, NHID), lambda i: (0, 0)),
            pl.BlockSpec((NMEM, NHID), lambda i: (0, 0)),
            pl.BlockSpec((2 * NHID * 2, NCLASS), lambda i: (0, 0)),
            pl.BlockSpec((1, NCLASS), lambda i: (0, 0)),
            pl.BlockSpec((NGRAPH, NHID), lambda i: (0, 0)),
            pl.BlockSpec((NGRAPH, NHID), lambda i: (0, 0)),
            pl.BlockSpec((NGRAPH, NHID), lambda i: (0, 0)),
        ],
        out_specs=pl.BlockSpec((NGRAPH, NCLASS), lambda i: (0, 0)),
        out_shape=jax.ShapeDtypeStruct((NGRAPH, NCLASS), jnp.float32),
        scratch_shapes=[
            pltpu.VMEM((NGRAPH, NHID), jnp.float32),
            pltpu.VMEM((NGRAPH, NHID), jnp.float32),
        ],
    )(s0p, q, q, batp, mem0, W1, b1, W2, b2, mem1, Wl, bl, max1, sum1, cnt1)


def kernel(x, edge_index, batch, W_pre, b_pre, W1_0, b1_0, W2_0, b2_0, mem_0,
           W1_1, b1_1, W2_1, b2_1, mem_1, W_lin, b_lin):
    edge2d = edge_index.reshape(2 * NCHUNK, 128)
    batp = batch.reshape(N, 1).astype(jnp.int32)

    h, h_bf = _tc_pre(x, W_pre, b_pre.reshape(1, NHID))
    p = _make_segsum(128, 128, jnp.bfloat16)(h_bf, edge2d)
    s0p = _tc_layer0(h, p, W1_0, b1_0.reshape(1, NHID),
                     W2_0, b2_0.reshape(1, NHID), mem_0)
    q = _make_segsum(16, 16)(s0p, edge2d)
    max1, sum1, cnt1 = _tc_pool1(s0p, batp, mem_0)
    return _tc_final(s0p, q, batp, mem_0,
                     W1_1, b1_1.reshape(1, NHID), W2_1, b2_1.reshape(1, NHID),
                     mem_1, W_lin, b_lin.reshape(1, NCLASS), max1, sum1, cnt1)


# speedup vs baseline: 1.0326x; 1.0326x over previous
import functools

import jax
import jax.numpy as jnp
from jax import lax
from jax.experimental import pallas as pl
from jax.experimental.pallas import tpu as pltpu
from jax.experimental.pallas import tpu_sc as plsc

N = 10000
E = 320000
NFEAT = 128
NHID = 128
NCLASS = 10
NMEM = 8
NGRAPH = 64

NP = 10240
NCHUNK = E // 128
KBASE = NCHUNK // 32
KREM = NCHUNK % 32
BLK = 400
NBLK = N // BLK

_NEG = -3.0e38


@functools.lru_cache(maxsize=None)
def _make_segsum(W, TW, dtype=jnp.float32):
    npass = TW // W
    rows_per_tile = NP // 16
    stage_rows = N // 16
    lanes = 16 * 4 // jnp.dtype(dtype).itemsize

    mesh = plsc.VectorSubcoreMesh(core_axis_name="c", subcore_axis_name="s")

    @functools.partial(
        pl.kernel,
        out_type=jax.ShapeDtypeStruct((npass, 2, NP, W), dtype),
        mesh=mesh,
        scratch_types=[
            pltpu.VMEM((KBASE + 1, 128), jnp.int32),
            pltpu.VMEM((KBASE + 1, 128), jnp.int32),
            pltpu.VMEM((2, 128, W), dtype),
            pltpu.VMEM_SHARED((NP, W), dtype),
            pltpu.VMEM_SHARED((NP, W), dtype),
            pltpu.SemaphoreType.DMA,
            pltpu.SemaphoreType.DMA,
        ],
        compiler_params=pltpu.CompilerParams(use_tc_tiling_on_sc=False),
    )
    def segsum(table_hbm, edge_hbm, out_hbm, src_v, dst_v, rows_v, tab_sh,
               acc_sh, sem0, sem1):
        cid = lax.axis_index("c")
        sid = lax.axis_index("s")
        tid = cid * 16 + sid
        base = tid * KBASE + jnp.minimum(tid, KREM)
        K = KBASE + jnp.where(tid < KREM, 1, 0)

        pltpu.sync_copy(edge_hbm.at[pl.ds(base, KBASE)],
                        src_v.at[pl.ds(0, KBASE)])
        pltpu.sync_copy(edge_hbm.at[pl.ds(NCHUNK + base, KBASE)],
                        dst_v.at[pl.ds(0, KBASE)])

        @pl.when(tid < KREM)
        def _extra():
            pltpu.sync_copy(edge_hbm.at[pl.ds(base + KBASE, 1)],
                            src_v.at[pl.ds(KBASE, 1)])
            pltpu.sync_copy(edge_hbm.at[pl.ds(NCHUNK + base + KBASE, 1)],
                            dst_v.at[pl.ds(KBASE, 1)])

        sems = (sem0, sem1)

        def _start(j, b):
            pltpu.async_copy(tab_sh.at[src_v.at[j]], rows_v.at[b], sems[b])

        def _wait(j, b):
            pltpu.make_async_copy(tab_sh.at[src_v.at[j]], rows_v.at[b],
                                  sems[b]).wait()

        def _scat(j, b):
            pltpu.sync_copy(rows_v.at[b], acc_sh.at[dst_v.at[j]], add=True)

        for p in range(npass):
            def _zrow(r, _):
                for j in range(W // lanes):
                    rows_v[0, r, pl.ds(j * lanes, lanes)] = jnp.zeros(
                        (lanes,), dtype)
                return 0
            lax.fori_loop(0, 128, _zrow, 0)
            for kk in range(rows_per_tile // 128):
                pltpu.sync_copy(
                    rows_v.at[0],
                    acc_sh.at[pl.ds(sid * rows_per_tile + kk * 128, 128)])
            pltpu.sync_copy(
                table_hbm.at[pl.ds(sid * stage_rows, stage_rows),
                             pl.ds(p * W, W)],
                tab_sh.at[pl.ds(sid * stage_rows, stage_rows)])
            plsc.subcore_barrier()

            _start(0, 0)

            def _body(i, _):
                jj = 2 * i
                _start(jj + 1, 1)
                _wait(jj, 0)
                _scat(jj, 0)

                @pl.when(jj + 2 < K)
                def _n0():
                    _start(jj + 2, 0)
                _wait(jj + 1, 1)
                _scat(jj + 1, 1)

                @pl.when(jj + 3 < K)
                def _n1():
                    _start(jj + 3, 1)
                return 0
            lax.fori_loop(0, KBASE // 2, _body, 0)

            @pl.when(K > KBASE)
            def _tail():
                _wait(KBASE, 0)
                _scat(KBASE, 0)

            plsc.subcore_barrier()
            pltpu.sync_copy(
                acc_sh.at[pl.ds(sid * rows_per_tile, rows_per_tile)],
                out_hbm.at[p, cid, pl.ds(sid * rows_per_tile, rows_per_tile)])

    return segsum


def _pre_body(x_ref, w_ref, b_ref, o_ref, ob_ref):
    r = jnp.dot(x_ref[...], w_ref[...],
                preferred_element_type=jnp.float32) + b_ref[...]
    o_ref[...] = r
    ob_ref[...] = r.astype(jnp.bfloat16)


def _tc_pre(x, W_pre, b_pre):
    return pl.pallas_call(
        _pre_body,
        grid=(NBLK,),
        in_specs=[
            pl.BlockSpec((BLK, NFEAT), lambda i: (i, 0)),
            pl.BlockSpec((NFEAT, NHID), lambda i: (0, 0)),
            pl.BlockSpec((1, NHID), lambda i: (0, 0)),
        ],
        out_specs=[pl.BlockSpec((BLK, NHID), lambda i: (i, 0)),
                   pl.BlockSpec((BLK, NHID), lambda i: (i, 0))],
        out_shape=[jax.ShapeDtypeStruct((N, NHID), jnp.float32),
                   jax.ShapeDtypeStruct((N, NHID), jnp.bfloat16)],
    )(x, W_pre, b_pre)


def _softmax8(t):
    m = jnp.max(t, axis=-1, keepdims=True)
    e = jnp.exp(t - m)
    return e / jnp.sum(e, axis=-1, keepdims=True)


def _layer0_body(h_ref, p0_ref, p1_ref,
                 w1_ref, b1_ref, w2_ref, b2_ref, mem_ref, o_ref):
    agg = (p0_ref[0, 0].astype(jnp.float32)
           + p1_ref[0, 0].astype(jnp.float32))
    z = jnp.concatenate([h_ref[...], agg], axis=1)
    z = jnp.maximum(jnp.dot(z, w1_ref[...],
                            preferred_element_type=jnp.float32) + b1_ref[...],
                    0.0)
    z = jnp.dot(z, w2_ref[...], preferred_element_type=jnp.float32) + b2_ref[...]
    z = jnp.maximum(z, 0.0)
    t = lax.dot_general(z, mem_ref[...], (((1,), (1,)), ((), ())),
                        preferred_element_type=jnp.float32)
    s = _softmax8(t)
    o_ref[...] = jnp.concatenate([s, jnp.zeros_like(s)], axis=1)


def _tc_layer0(h, p, W1, b1, W2, b2, mem):
    return pl.pallas_call(
        _layer0_body,
        grid=(NBLK,),
        in_specs=[
            pl.BlockSpec((BLK, NHID), lambda i: (i, 0)),
            pl.BlockSpec((1, 1, BLK, NHID), lambda i: (0, 0, i, 0)),
            pl.BlockSpec((1, 1, BLK, NHID), lambda i: (0, 1, i, 0)),
            pl.BlockSpec((2 * NHID, NHID), lambda i: (0, 0)),
            pl.BlockSpec((1, NHID), lambda i: (0, 0)),
            pl.BlockSpec((NHID, NHID), lambda i: (0, 0)),
            pl.BlockSpec((1, NHID), lambda i: (0, 0)),
            pl.BlockSpec((NMEM, NHID), lambda i: (0, 0)),
        ],
        out_specs=pl.BlockSpec((BLK, 16), lambda i: (i, 0)),
        out_shape=jax.ShapeDtypeStruct((N, 16), jnp.float32),
    )(h, p, p, W1, b1, W2, b2, mem)


def _final_body(s0_ref, q0_ref, q1_ref, bat_ref, mem0_ref, w1_ref, b1_ref,
                w2_ref, b2_ref, mem1_ref, wl_ref, bl_ref, o_ref,
                max_ref, sum_ref, cnt_ref):
    i = pl.program_id(0)

    @pl.when(i == 0)
    def _init():
        max_ref[...] = jnp.full_like(max_ref, _NEG)
        sum_ref[...] = jnp.zeros_like(sum_ref)
        cnt_ref[...] = jnp.zeros_like(cnt_ref)

    s0 = s0_ref[...][:, :NMEM]
    a8 = (q0_ref[0, 0] + q1_ref[0, 0])[:, :NMEM]
    mem0 = mem0_ref[...]
    h1 = jnp.dot(s0, mem0, preferred_element_type=jnp.float32)
    agg1 = jnp.dot(a8, mem0, preferred_element_type=jnp.float32)
    z = jnp.concatenate([h1, agg1], axis=1)
    z = jnp.maximum(jnp.dot(z, w1_ref[...],
                            preferred_element_type=jnp.float32) + b1_ref[...],
                    0.0)
    z = jnp.dot(z, w2_ref[...], preferred_element_type=jnp.float32) + b2_ref[...]
    z = jnp.maximum(z, 0.0)
    t = lax.dot_general(z, mem1_ref[...], (((1,), (1,)), ((), ())),
                        preferred_element_type=jnp.float32)
    s1 = _softmax8(t)
    h2 = jnp.dot(s1, mem1_ref[...], preferred_element_type=jnp.float32)
    hcat = jnp.concatenate([h1, h2], axis=1)

    bat = bat_ref[...]
    Pf = (lax.broadcasted_iota(jnp.int32, (BLK, NGRAPH), 1) == bat
          ).astype(jnp.float32)
    sum_ref[...] += lax.dot_general(Pf, hcat, (((0,), (0,)), ((), ())),
                                    preferred_element_type=jnp.float32)
    cnt_ref[...] += lax.dot_general(
        Pf, jnp.ones((BLK, 128), jnp.float32), (((0,), (0,)), ((), ())),
        preferred_element_type=jnp.float32)

    gmin = jnp.min(bat)
    gmax = jnp.max(bat)

    def _upd(g, _):
        m = jnp.max(jnp.where(bat == g, hcat, _NEG), axis=0, keepdims=True)
        max_ref[pl.ds(g, 1), :] = jnp.maximum(max_ref[pl.ds(g, 1), :], m)
        return 0
    lax.fori_loop(gmin, gmax + 1, _upd, 0)

    @pl.when(i == NBLK - 1)
    def _fin():
        out1 = max_ref[...]
        out1 = jnp.where(out1 > _NEG * 0.5, out1, 0.0)
        cnt = cnt_ref[...][:, 0:1]
        out2 = sum_ref[...] / jnp.maximum(cnt, 1.0)
        gfeat = jnp.concatenate([out1, out2], axis=1)
        logits = jnp.dot(gfeat, wl_ref[...],
                         preferred_element_type=jnp.float32) + bl_ref[...]
        m = jnp.max(logits, axis=-1, keepdims=True)
        lse = m + jnp.log(jnp.sum(jnp.exp(logits - m), axis=-1, keepdims=True))
        o_ref[...] = logits - lse


def _tc_final(s0p, q, batp, mem0, W1, b1, W2, b2, mem1, Wl, bl):
    return pl.pallas_call(
        _final_body,
        grid=(NBLK,),
        in_specs=[
            pl.BlockSpec((BLK, 16), lambda i: (i, 0)),
            pl.BlockSpec((1, 1, BLK, 16), lambda i: (0, 0, i, 0)),
            pl.BlockSpec((1, 1, BLK, 16), lambda i: (0, 1, i, 0)),
            pl.BlockSpec((BLK, 1), lambda i: (i, 0)),
            pl.BlockSpec((NMEM, NHID), lambda i: (0, 0)),
            pl.BlockSpec((2 * NHID, NHID), lambda i: (0, 0)),
            pl.BlockSpec((1, NHID), lambda i: (0, 0)),
            pl.BlockSpec((NHID, NHID), lambda i: (0, 0)),
            pl.BlockSpec((1, NHID), lambda i: (0, 0)),
            pl.BlockSpec((NMEM, NHID), lambda i: (0, 0)),
            pl.BlockSpec((2 * NHID * 2, NCLASS), lambda i: (0, 0)),
            pl.BlockSpec((1, NCLASS), lambda i: (0, 0)),
        ],
        out_specs=pl.BlockSpec((NGRAPH, NCLASS), lambda i: (0, 0)),
        out_shape=jax.ShapeDtypeStruct((NGRAPH, NCLASS), jnp.float32),
        scratch_shapes=[
            pltpu.VMEM((NGRAPH, 256), jnp.float32),
            pltpu.VMEM((NGRAPH, 256), jnp.float32),
            pltpu.VMEM((NGRAPH, 128), jnp.float32),
        ],
    )(s0p, q, q, batp, mem0, W1, b1, W2, b2, mem1, Wl, bl)


def kernel(x, edge_index, batch, W_pre, b_pre, W1_0, b1_0, W2_0, b2_0, mem_0,
           W1_1, b1_1, W2_1, b2_1, mem_1, W_lin, b_lin):
    edge2d = edge_index.reshape(2 * NCHUNK, 128)
    batp = batch.reshape(N, 1).astype(jnp.int32)

    h, h_bf = _tc_pre(x, W_pre, b_pre.reshape(1, NHID))
    p = _make_segsum(128, 128, jnp.bfloat16)(h_bf, edge2d)
    s0p = _tc_layer0(h, p, W1_0, b1_0.reshape(1, NHID),
                     W2_0, b2_0.reshape(1, NHID), mem_0)
    q = _make_segsum(16, 16)(s0p, edge2d)
    return _tc_final(s0p, q, batp, mem_0,
                     W1_1, b1_1.reshape(1, NHID), W2_1, b2_1.reshape(1, NHID),
                     mem_1, W_lin, b_lin.reshape(1, NCLASS))
